# baseline (speedup 1.0000x reference)
import functools

import jax
import jax.numpy as jnp
from jax import lax
from jax.experimental import pallas as pl
from jax.experimental.pallas import tpu as pltpu
from jax.experimental.pallas import tpu_sc as plsc


def _make_gather(V, D, B):
    info = plsc.get_sparse_core_info()
    NC, NS = info.num_cores, info.num_subcores
    NW = NC * NS
    assert B % (8 * NW) == 0
    b_per_w = B // NW
    CHUNK = 16
    NCHUNK = b_per_w // CHUNK
    NSUPER = 3
    PUT_W = 2 * CHUNK
    NPUT = b_per_w // PUT_W
    mesh = plsc.VectorSubcoreMesh(core_axis_name="c", subcore_axis_name="s")

    @functools.partial(
        pl.kernel,
        mesh=mesh,
        out_type=jax.ShapeDtypeStruct((B, D), jnp.float32),
        scratch_types=[
            pltpu.VMEM((b_per_w,), jnp.int32),
            pltpu.VMEM((NSUPER, 2 * CHUNK, D), jnp.float32),
            pltpu.SemaphoreType.DMA((NSUPER, 2)),
            pltpu.SemaphoreType.DMA((NSUPER,)),
        ],
    )
    def k(table_hbm, idx_hbm, out_hbm, idx_v, rows_v, gsem, osem):
        wid = lax.axis_index("s") * NC + lax.axis_index("c")
        base = wid * b_per_w
        pltpu.sync_copy(idx_hbm.at[pl.ds(base, b_per_w)], idx_v)

        def gather(c):
            sup = (c // 2) % NSUPER
            half = c % 2
            return pltpu.async_copy(
                table_hbm.at[idx_v.at[pl.ds(c * CHUNK, CHUNK)]],
                rows_v.at[sup, pl.ds(half * CHUNK, CHUNK)],
                gsem.at[sup, half],
            )

        def put(s):
            sup = s % NSUPER
            return pltpu.async_copy(
                rows_v.at[sup],
                out_hbm.at[pl.ds(base + s * PUT_W, PUT_W)],
                osem.at[sup],
            )

        gathers = [None] * NCHUNK
        puts = [None] * NPUT
        put_done = [False] * NPUT
        for c in range(min(4, NCHUNK)):
            gathers[c] = gather(c)
        for s in range(NPUT):
            gathers[2 * s].wait()
            gathers[2 * s + 1].wait()
            puts[s] = put(s)
            if s + 2 < NPUT:
                if s - 1 >= 0:
                    puts[s - 1].wait()
                    put_done[s - 1] = True
                gathers[2 * s + 4] = gather(2 * s + 4)
                gathers[2 * s + 5] = gather(2 * s + 5)
        for s in range(NPUT):
            if not put_done[s]:
                puts[s].wait()

    return k


def kernel(inputs, table):
    Bt, S = inputs.shape
    V, D = table.shape
    flat_idx = inputs.reshape(-1).astype(jnp.int32)
    out = _make_gather(V, D, Bt * S)(table, flat_idx)
    return out.reshape(Bt, S, D)


# speedup vs baseline: 1.0470x; 1.0470x over previous
import functools

import jax
import jax.numpy as jnp
from jax import lax
from jax.experimental import pallas as pl
from jax.experimental.pallas import tpu as pltpu
from jax.experimental.pallas import tpu_sc as plsc


def _make_gather(V, D, Bt, S):
    info = plsc.get_sparse_core_info()
    NC, NS = info.num_cores, info.num_subcores
    NW = NC * NS
    B = Bt * S
    assert B % (8 * NW) == 0
    b_per_w = B // NW
    assert S % b_per_w == 0
    w_per_row = S // b_per_w
    CHUNK = 16
    NCHUNK = b_per_w // CHUNK
    NBUF = 6
    DEPTH = NBUF - 1
    mesh = plsc.VectorSubcoreMesh(core_axis_name="c", subcore_axis_name="s")

    @functools.partial(
        pl.kernel,
        mesh=mesh,
        out_type=jax.ShapeDtypeStruct((B, D), jnp.float32),
        scratch_types=[
            pltpu.VMEM((b_per_w,), jnp.int32),
            pltpu.VMEM((NBUF, CHUNK, D), jnp.float32),
            pltpu.SemaphoreType.DMA((NBUF,)),
            pltpu.SemaphoreType.DMA((NBUF,)),
        ],
    )
    def k(table_hbm, idx_hbm, out_hbm, idx_v, rows_v, gsem, osem):
        wid = lax.axis_index("s") * NC + lax.axis_index("c")
        base = wid * b_per_w
        row = wid // w_per_row
        col = (wid % w_per_row) * b_per_w
        pltpu.sync_copy(idx_hbm.at[row, pl.ds(col, b_per_w)], idx_v)

        def gather(c):
            buf = c % NBUF
            return pltpu.async_copy(
                table_hbm.at[idx_v.at[pl.ds(c * CHUNK, CHUNK)]],
                rows_v.at[buf],
                gsem.at[buf],
            )

        def put(c):
            buf = c % NBUF
            return pltpu.async_copy(
                rows_v.at[buf],
                out_hbm.at[pl.ds(base + c * CHUNK, CHUNK)],
                osem.at[buf],
            )

        gathers = [None] * NCHUNK
        puts = [None] * NCHUNK
        put_done = [False] * NCHUNK
        for c in range(min(DEPTH, NCHUNK)):
            gathers[c] = gather(c)
        for c in range(NCHUNK):
            gathers[c].wait()
            puts[c] = put(c)
            if c + DEPTH < NCHUNK:
                j = c + DEPTH - NBUF
                if j >= 0:
                    puts[j].wait()
                    put_done[j] = True
                gathers[c + DEPTH] = gather(c + DEPTH)
        for c in range(NCHUNK):
            if not put_done[c]:
                puts[c].wait()

    return k


def kernel(inputs, table):
    Bt, S = inputs.shape
    V, D = table.shape
    out = _make_gather(V, D, Bt, S)(table, inputs)
    return out.reshape(Bt, S, D)


# speedup vs baseline: 1.0476x; 1.0006x over previous
import functools

import jax
import jax.numpy as jnp
from jax import lax
from jax.experimental import pallas as pl
from jax.experimental.pallas import tpu as pltpu
from jax.experimental.pallas import tpu_sc as plsc


def _make_gather(V, D, Bt, S):
    info = plsc.get_sparse_core_info()
    NC, NS = info.num_cores, info.num_subcores
    NW = NC * NS
    B = Bt * S
    assert B % (8 * NW) == 0
    b_per_w = B // NW
    assert S % b_per_w == 0
    w_per_row = S // b_per_w
    CHUNK = 16
    NCHUNK = b_per_w // CHUNK
    NBUF = 6
    DEPTH = NBUF - 1
    mesh = plsc.VectorSubcoreMesh(core_axis_name="c", subcore_axis_name="s")

    @functools.partial(
        pl.kernel,
        mesh=mesh,
        out_type=jax.ShapeDtypeStruct((B, D), jnp.float32),
        scratch_types=[
            pltpu.VMEM((b_per_w,), jnp.int32),
            pltpu.VMEM((NBUF, CHUNK, D), jnp.float32),
            pltpu.SemaphoreType.DMA((NBUF,)),
            pltpu.SemaphoreType.DMA((NBUF,)),
        ],
    )
    def k(table_hbm, idx_hbm, out_hbm, idx_v, rows_v, gsem, osem):
        wid = lax.axis_index("s") * NC + lax.axis_index("c")
        base = wid * b_per_w
        row = wid // w_per_row
        col = (wid % w_per_row) * b_per_w
        pltpu.sync_copy(idx_hbm.at[row, pl.ds(col, b_per_w)], idx_v)

        def gather(c):
            buf = c % NBUF
            return pltpu.async_copy(
                table_hbm.at[idx_v.at[pl.ds(c * CHUNK, CHUNK)]],
                rows_v.at[buf],
                gsem.at[buf],
            )

        def put(c):
            buf = c % NBUF
            return pltpu.async_copy(
                rows_v.at[buf],
                out_hbm.at[pl.ds(base + c * CHUNK, CHUNK)],
                osem.at[buf],
            )

        gathers = [None] * NCHUNK
        puts = [None] * NCHUNK
        put_done = [False] * NCHUNK
        for c in range(min(DEPTH, NCHUNK)):
            gathers[c] = gather(c)
        for c in range(NCHUNK):
            gathers[c].wait()
            if c + DEPTH < NCHUNK:
                j = c + DEPTH - NBUF
                if j >= 0:
                    puts[j].wait()
                    put_done[j] = True
                gathers[c + DEPTH] = gather(c + DEPTH)
            puts[c] = put(c)
        for c in range(NCHUNK):
            if not put_done[c]:
                puts[c].wait()

    return k


def kernel(inputs, table):
    Bt, S = inputs.shape
    V, D = table.shape
    out = _make_gather(V, D, Bt, S)(table, inputs)
    return out.reshape(Bt, S, D)
